# Initial kernel scaffold; baseline (speedup 1.0000x reference)
import jax
import jax.numpy as jnp
from jax.experimental import pallas as pl


def kernel(x, Wr, Wg, Wu, Wd):
    raise NotImplementedError("write your pallas kernel here")


# speedup vs baseline: 1.0070x; 1.0070x over previous
import functools

import jax
import jax.numpy as jnp
from jax import lax
from jax.experimental import pallas as pl
from jax.experimental.pallas import tpu as pltpu
from jax.experimental.pallas import tpu_sc as plsc

T = 2048
D = 768
H = 2048
E = 8
K = 2
N = T * K

BM = 128
NB = N // BM
NTILES = NB + E - 1

RB = 512
NC = 2
NS = 16
NW = NC * NS
BPW = N // NW
TPW = T // NW


def _router_body(x_ref, wr_ref, e1_ref, e2_ref, w1_ref, w2_ref):
    xb = x_ref[...]
    logits = lax.dot_general(xb, wr_ref[...], (((1,), (1,)), ((), ())),
                             preferred_element_type=jnp.float32)
    p = jax.nn.softmax(logits, axis=-1)
    col = lax.broadcasted_iota(jnp.int32, p.shape, 1)
    m1 = jnp.max(p, axis=-1)
    i1 = jnp.min(jnp.where(p == m1[:, None], col, E), axis=-1)
    p2 = jnp.where(col == i1[:, None], -1.0, p)
    m2 = jnp.max(p2, axis=-1)
    i2 = jnp.min(jnp.where(p2 == m2[:, None], col, E), axis=-1)
    den = m1 + m2
    e1_ref[...] = i1[:, None]
    e2_ref[...] = i2[:, None]
    w1_ref[...] = (m1 / den)[:, None]
    w2_ref[...] = (m2 / den)[:, None]


def _route(x2d, Wr):
    return pl.pallas_call(
        _router_body,
        grid=(T // RB,),
        in_specs=[pl.BlockSpec((RB, D), lambda r: (r, 0)),
                  pl.BlockSpec((E, D), lambda r: (0, 0))],
        out_specs=[pl.BlockSpec((RB, 1), lambda r: (r, 0))] * 4,
        out_shape=[jax.ShapeDtypeStruct((T, 1), jnp.int32),
                   jax.ShapeDtypeStruct((T, 1), jnp.int32),
                   jax.ShapeDtypeStruct((T, 1), jnp.float32),
                   jax.ShapeDtypeStruct((T, 1), jnp.float32)],
    )(x2d, Wr)


@functools.cache
def _get_sc_dispatch():
    mesh = plsc.VectorSubcoreMesh(core_axis_name="c", subcore_axis_name="s")

    @functools.partial(
        pl.kernel,
        mesh=mesh,
        out_type=jax.ShapeDtypeStruct((N, D), jnp.float32),
        scratch_types=[pltpu.VMEM((BPW,), jnp.int32),
                       pltpu.VMEM((BPW, D), jnp.float32),
                       pltpu.SemaphoreType.DMA],
    )
    def _sc_dispatch(x_hbm, idx_hbm, out_hbm, idx_v, rows_v, sem):
        wid = lax.axis_index("s") * NC + lax.axis_index("c")
        base = wid * BPW
        pltpu.sync_copy(idx_hbm.at[pl.ds(base, BPW)], idx_v)
        pltpu.async_copy(x_hbm.at[idx_v], rows_v, sem).wait()
        pltpu.sync_copy(rows_v, out_hbm.at[pl.ds(base, BPW)])

    return _sc_dispatch


@functools.cache
def _get_sc_combine():
    mesh = plsc.VectorSubcoreMesh(core_axis_name="c", subcore_axis_name="s")

    @functools.partial(
        pl.kernel,
        mesh=mesh,
        out_type=jax.ShapeDtypeStruct((T, D), jnp.float32),
        scratch_types=[pltpu.VMEM((TPW,), jnp.int32),
                       pltpu.VMEM((TPW,), jnp.int32),
                       pltpu.VMEM((TPW, D), jnp.float32),
                       pltpu.VMEM((TPW, D), jnp.float32),
                       pltpu.SemaphoreType.DMA,
                       pltpu.SemaphoreType.DMA],
    )
    def _sc_combine(ys_hbm, ia_hbm, ib_hbm, out_hbm, ia_v, ib_v, ra_v, rb_v,
                    sa, sb):
        wid = lax.axis_index("s") * NC + lax.axis_index("c")
        base = wid * TPW
        pltpu.sync_copy(ia_hbm.at[pl.ds(base, TPW)], ia_v)
        pltpu.sync_copy(ib_hbm.at[pl.ds(base, TPW)], ib_v)
        ca = pltpu.async_copy(ys_hbm.at[ia_v], ra_v, sa)
        cb = pltpu.async_copy(ys_hbm.at[ib_v], rb_v, sb)
        ca.wait()
        cb.wait()

        def _row(j, carry):
            def _col(c, carry2):
                sl = pl.ds(c * 16, 16)
                ra_v[j, sl] = ra_v[j, sl] + rb_v[j, sl]
                return carry2
            return lax.fori_loop(0, D // 16, _col, carry, unroll=8)

        lax.fori_loop(0, TPW, _row, 0)
        pltpu.sync_copy(ra_v, out_hbm.at[pl.ds(base, TPW)])

    return _sc_combine


def _expert_body(tb_ref, tg_ref, off_ref, xs_ref, wg_ref, wu_ref, wd_ref,
                 wr_ref, ys_ref):
    i = pl.program_id(0)
    g = tg_ref[i]
    b = tb_ref[i]
    lo = off_ref[g]
    hi = off_ref[g + 1]
    xb = xs_ref[...]
    gate = lax.dot_general(xb, wg_ref[0], (((1,), (1,)), ((), ())),
                           preferred_element_type=jnp.float32)
    up = lax.dot_general(xb, wu_ref[0], (((1,), (1,)), ((), ())),
                         preferred_element_type=jnp.float32)
    act = jax.nn.silu(gate) * up
    y = lax.dot_general(act, wd_ref[0], (((1,), (1,)), ((), ())),
                        preferred_element_type=jnp.float32)
    y = y * wr_ref[0]
    rows = b * BM + lax.broadcasted_iota(jnp.int32, (BM, 1), 0)
    mask = (rows >= lo) & (rows < hi)
    ys_ref[...] = jnp.where(mask, y, ys_ref[...])


def _grouped_experts(tile_b, tile_g, offsets, xs, Wg, Wu, Wd, wr3):
    return pl.pallas_call(
        _expert_body,
        grid_spec=pltpu.PrefetchScalarGridSpec(
            num_scalar_prefetch=3,
            grid=(NTILES,),
            in_specs=[
                pl.BlockSpec((BM, D), lambda i, tb, tg, off: (tb[i], 0)),
                pl.BlockSpec((1, H, D), lambda i, tb, tg, off: (tg[i], 0, 0)),
                pl.BlockSpec((1, H, D), lambda i, tb, tg, off: (tg[i], 0, 0)),
                pl.BlockSpec((1, D, H), lambda i, tb, tg, off: (tg[i], 0, 0)),
                pl.BlockSpec((1, BM, 1), lambda i, tb, tg, off: (tb[i], 0, 0)),
            ],
            out_specs=pl.BlockSpec((BM, D), lambda i, tb, tg, off: (tb[i], 0)),
        ),
        out_shape=jax.ShapeDtypeStruct((N, D), jnp.float32),
    )(tile_b, tile_g, offsets, xs, Wg, Wu, Wd, wr3)


def kernel(x, Wr, Wg, Wu, Wd):
    x2d = x.reshape(T, D)
    e1, e2, w1, w2 = _route(x2d, Wr)
    e1 = e1.reshape(T)
    e2 = e2.reshape(T)
    w1 = w1.reshape(T)
    w2 = w2.reshape(T)

    e_pairs = jnp.stack([e1, e2], axis=1).reshape(N)
    w_pairs = jnp.stack([w1, w2], axis=1).reshape(N)
    onehot = (e_pairs[:, None] == jnp.arange(E, dtype=jnp.int32)[None, :])
    csum = jnp.cumsum(onehot.astype(jnp.int32), axis=0)
    counts = csum[-1]
    offsets = jnp.concatenate(
        [jnp.zeros(1, jnp.int32), jnp.cumsum(counts)]).astype(jnp.int32)
    rank = jnp.take_along_axis(csum, e_pairs[:, None], axis=1)[:, 0] - 1
    pos = offsets[e_pairs] + rank
    order = jnp.zeros(N, jnp.int32).at[pos].set(
        jnp.arange(N, dtype=jnp.int32))
    st = order // K
    wr = w_pairs[order]

    fb = offsets[:E] // BM
    lb = (offsets[1:] - 1) // BM
    s = jnp.where(counts > 0, lb - fb + 1, 0)
    ws = jnp.concatenate([jnp.zeros(1, jnp.int32),
                          jnp.cumsum(s)]).astype(jnp.int32)
    wtot = ws[E]
    i_arr = jnp.arange(NTILES, dtype=jnp.int32)
    g_raw = jnp.sum((i_arr[:, None] >= ws[None, 1:]).astype(jnp.int32), axis=1)
    g_c = jnp.minimum(g_raw, E - 1)
    b_raw = fb[g_c] + (i_arr - ws[g_c])
    valid = i_arr < wtot
    tile_g = jnp.where(valid, g_c, jnp.take(g_c, wtot - 1)).astype(jnp.int32)
    tile_b = jnp.where(valid, b_raw, jnp.take(b_raw, wtot - 1)).astype(jnp.int32)

    xs = _get_sc_dispatch()(x2d, st.astype(jnp.int32))
    wr3 = wr.reshape(NB, BM, 1)
    ys = _grouped_experts(tile_b, tile_g, offsets, xs, Wg, Wu, Wd, wr3)
    inv2 = pos.reshape(T, K)
    out = _get_sc_combine()(ys, inv2[:, 0].astype(jnp.int32),
                            inv2[:, 1].astype(jnp.int32))
    return out.reshape(1, T, D)


# speedup vs baseline: 1.1700x; 1.1619x over previous
import functools

import jax
import jax.numpy as jnp
from jax import lax
from jax.experimental import pallas as pl
from jax.experimental.pallas import tpu as pltpu
from jax.experimental.pallas import tpu_sc as plsc

T = 2048
D = 768
H = 2048
E = 8
K = 2
N = T * K

BM = 128
NB = N // BM
NTILES = NB + E - 1

NC = 2
NS = 16
NW = NC * NS
TPW = T // NW
WL = 128


def _router_body(x_ref, wr_ref, p1_ref, p2_ref, w1b_ref, w2b_ref, off_ref):
    xb = x_ref[...]
    logits = lax.dot_general(xb, wr_ref[...], (((1,), (1,)), ((), ())),
                             preferred_element_type=jnp.float32)
    p = jax.nn.softmax(logits, axis=-1)
    col = lax.broadcasted_iota(jnp.int32, p.shape, 1)
    m1 = jnp.max(p, axis=-1)
    i1 = jnp.min(jnp.where(p == m1[:, None], col, E), axis=-1)
    pm = jnp.where(col == i1[:, None], -1.0, p)
    m2 = jnp.max(pm, axis=-1)
    i2 = jnp.min(jnp.where(pm == m2[:, None], col, E), axis=-1)
    den = m1 + m2
    w1 = m1 / den
    w2 = m2 / den

    oh = ((col == i1[:, None]) | (col == i2[:, None])).astype(jnp.float32)
    s = oh
    k = 1
    while k < T:
        s = s + jnp.concatenate([jnp.zeros((k, E), jnp.float32), s[:-k]], 0)
        k *= 2
    cex = s - oh
    total = s[T - 1]
    erow = lax.broadcasted_iota(jnp.int32, (E, E), 0)
    ecol = lax.broadcasted_iota(jnp.int32, (E, E), 1)
    off = jnp.sum(jnp.where(erow < ecol, total[:, None], 0.0), axis=0)

    def _sel(mat, idx):
        return jnp.sum(jnp.where(col == idx[:, None], mat, 0.0), axis=1)

    offb = jnp.broadcast_to(off[None, :], (T, E))
    pos1 = _sel(offb, i1) + _sel(cex, i1)
    pos2 = _sel(offb, i2) + _sel(cex, i2)

    p1_ref[...] = pos1.astype(jnp.int32)[:, None]
    p2_ref[...] = pos2.astype(jnp.int32)[:, None]
    w1b_ref[...] = jnp.broadcast_to(w1[:, None], (T, WL))
    w2b_ref[...] = jnp.broadcast_to(w2[:, None], (T, WL))
    off9 = jnp.concatenate([off, jnp.full((8,), float(N), jnp.float32)])
    off_ref[...] = off9.astype(jnp.int32)[:, None]


def _route(x2d, Wr):
    return pl.pallas_call(
        _router_body,
        grid=(1,),
        in_specs=[pl.BlockSpec((T, D), lambda i: (0, 0)),
                  pl.BlockSpec((E, D), lambda i: (0, 0))],
        out_specs=[pl.BlockSpec((T, 1), lambda i: (0, 0)),
                   pl.BlockSpec((T, 1), lambda i: (0, 0)),
                   pl.BlockSpec((T, WL), lambda i: (0, 0)),
                   pl.BlockSpec((T, WL), lambda i: (0, 0)),
                   pl.BlockSpec((16, 1), lambda i: (0, 0))],
        out_shape=[jax.ShapeDtypeStruct((T, 1), jnp.int32),
                   jax.ShapeDtypeStruct((T, 1), jnp.int32),
                   jax.ShapeDtypeStruct((T, WL), jnp.float32),
                   jax.ShapeDtypeStruct((T, WL), jnp.float32),
                   jax.ShapeDtypeStruct((16, 1), jnp.int32)],
    )(x2d, Wr)


@functools.cache
def _get_sc_dispatch():
    mesh = plsc.VectorSubcoreMesh(core_axis_name="c", subcore_axis_name="s")

    @functools.partial(
        pl.kernel,
        mesh=mesh,
        out_type=[jax.ShapeDtypeStruct((N, D), jnp.float32),
                  jax.ShapeDtypeStruct((N, WL), jnp.float32)],
        scratch_types=[pltpu.VMEM((TPW, D), jnp.float32),
                       pltpu.VMEM((TPW,), jnp.int32),
                       pltpu.VMEM((TPW,), jnp.int32),
                       pltpu.VMEM((TPW, WL), jnp.float32),
                       pltpu.VMEM((TPW, WL), jnp.float32),
                       pltpu.SemaphoreType.DMA,
                       pltpu.SemaphoreType.DMA,
                       pltpu.SemaphoreType.DMA,
                       pltpu.SemaphoreType.DMA],
    )
    def _sc_dispatch(x_hbm, p1_hbm, p2_hbm, w1b_hbm, w2b_hbm, xs_hbm, wr_hbm,
                     xv, i1v, i2v, wv1, wv2, s1, s2, s3, s4):
        wid = lax.axis_index("s") * NC + lax.axis_index("c")
        base = wid * TPW
        pltpu.sync_copy(x_hbm.at[pl.ds(base, TPW)], xv)
        pltpu.sync_copy(p1_hbm.at[pl.ds(base, TPW)], i1v)
        pltpu.sync_copy(p2_hbm.at[pl.ds(base, TPW)], i2v)
        pltpu.sync_copy(w1b_hbm.at[pl.ds(base, TPW)], wv1)
        pltpu.sync_copy(w2b_hbm.at[pl.ds(base, TPW)], wv2)
        c1 = pltpu.async_copy(xv, xs_hbm.at[i1v], s1)
        c2 = pltpu.async_copy(xv, xs_hbm.at[i2v], s2)
        c3 = pltpu.async_copy(wv1, wr_hbm.at[i1v], s3)
        c4 = pltpu.async_copy(wv2, wr_hbm.at[i2v], s4)
        c1.wait()
        c2.wait()
        c3.wait()
        c4.wait()

    return _sc_dispatch


@functools.cache
def _get_sc_combine():
    mesh = plsc.VectorSubcoreMesh(core_axis_name="c", subcore_axis_name="s")

    @functools.partial(
        pl.kernel,
        mesh=mesh,
        out_type=jax.ShapeDtypeStruct((T, D), jnp.float32),
        scratch_types=[pltpu.VMEM((TPW,), jnp.int32),
                       pltpu.VMEM((TPW,), jnp.int32),
                       pltpu.VMEM((TPW, D), jnp.float32),
                       pltpu.VMEM((TPW, D), jnp.float32),
                       pltpu.SemaphoreType.DMA,
                       pltpu.SemaphoreType.DMA],
    )
    def _sc_combine(ys_hbm, ia_hbm, ib_hbm, out_hbm, ia_v, ib_v, ra_v, rb_v,
                    sa, sb):
        wid = lax.axis_index("s") * NC + lax.axis_index("c")
        base = wid * TPW
        pltpu.sync_copy(ia_hbm.at[pl.ds(base, TPW)], ia_v)
        pltpu.sync_copy(ib_hbm.at[pl.ds(base, TPW)], ib_v)
        ca = pltpu.async_copy(ys_hbm.at[ia_v], ra_v, sa)
        cb = pltpu.async_copy(ys_hbm.at[ib_v], rb_v, sb)
        ca.wait()
        cb.wait()

        def _row(j, carry):
            def _col(c, carry2):
                sl = pl.ds(c * 16, 16)
                ra_v[j, sl] = ra_v[j, sl] + rb_v[j, sl]
                return carry2
            return lax.fori_loop(0, D // 16, _col, carry, unroll=8)

        lax.fori_loop(0, TPW, _row, 0)
        pltpu.sync_copy(ra_v, out_hbm.at[pl.ds(base, TPW)])

    return _sc_combine


def _expert_body(tb_ref, tg_ref, off_ref, xs_ref, wg_ref, wu_ref, wd_ref,
                 wr_ref, ys_ref):
    i = pl.program_id(0)
    g = tg_ref[i]
    b = tb_ref[i]
    lo = off_ref[g]
    hi = off_ref[g + 1]
    xb = xs_ref[...]
    gate = lax.dot_general(xb, wg_ref[0], (((1,), (1,)), ((), ())),
                           preferred_element_type=jnp.float32)
    up = lax.dot_general(xb, wu_ref[0], (((1,), (1,)), ((), ())),
                         preferred_element_type=jnp.float32)
    act = jax.nn.silu(gate) * up
    y = lax.dot_general(act, wd_ref[0], (((1,), (1,)), ((), ())),
                        preferred_element_type=jnp.float32)
    y = y * wr_ref[0][:, 0:1]
    rows = b * BM + lax.broadcasted_iota(jnp.int32, (BM, 1), 0)
    mask = (rows >= lo) & (rows < hi)
    ys_ref[...] = jnp.where(mask, y, ys_ref[...])


def _grouped_experts(tile_b, tile_g, offsets, xs, Wg, Wu, Wd, wr3):
    return pl.pallas_call(
        _expert_body,
        grid_spec=pltpu.PrefetchScalarGridSpec(
            num_scalar_prefetch=3,
            grid=(NTILES,),
            in_specs=[
                pl.BlockSpec((BM, D), lambda i, tb, tg, off: (tb[i], 0)),
                pl.BlockSpec((1, H, D), lambda i, tb, tg, off: (tg[i], 0, 0)),
                pl.BlockSpec((1, H, D), lambda i, tb, tg, off: (tg[i], 0, 0)),
                pl.BlockSpec((1, D, H), lambda i, tb, tg, off: (tg[i], 0, 0)),
                pl.BlockSpec((1, BM, WL), lambda i, tb, tg, off: (tb[i], 0, 0)),
            ],
            out_specs=pl.BlockSpec((BM, D), lambda i, tb, tg, off: (tb[i], 0)),
        ),
        out_shape=jax.ShapeDtypeStruct((N, D), jnp.float32),
    )(tile_b, tile_g, offsets, xs, Wg, Wu, Wd, wr3)


def kernel(x, Wr, Wg, Wu, Wd):
    x2d = x.reshape(T, D)
    pos1, pos2, w1b, w2b, off16 = _route(x2d, Wr)
    p1 = pos1.reshape(T)
    p2 = pos2.reshape(T)
    off16 = off16.reshape(16)

    off9 = off16[:9]
    counts = off9[1:] - off9[:8]
    fb = off9[:8] // BM
    lb = (off9[1:9] - 1) // BM
    s = jnp.where(counts > 0, lb - fb + 1, 0)
    ws = jnp.concatenate([jnp.zeros(1, jnp.int32),
                          jnp.cumsum(s)]).astype(jnp.int32)
    wtot = ws[E]
    i_arr = jnp.arange(NTILES, dtype=jnp.int32)
    g_raw = jnp.sum((i_arr[:, None] >= ws[None, 1:]).astype(jnp.int32), axis=1)
    g_c = jnp.minimum(g_raw, E - 1)
    b_raw = fb[g_c] + (i_arr - ws[g_c])
    valid = i_arr < wtot
    tile_g = jnp.where(valid, g_c, jnp.take(g_c, wtot - 1)).astype(jnp.int32)
    tile_b = jnp.where(valid, b_raw, jnp.take(b_raw, wtot - 1)).astype(jnp.int32)

    xs, wrD = _get_sc_dispatch()(x2d, p1, p2, w1b, w2b)
    ys = _grouped_experts(tile_b, tile_g, off16, xs, Wg, Wu, Wd,
                          wrD.reshape(NB, BM, WL))
    out = _get_sc_combine()(ys, p1, p2)
    return out.reshape(1, T, D)
